# baseline (speedup 1.0000x reference)
import functools

import jax
import jax.numpy as jnp
from jax import lax
from jax.experimental import pallas as pl
from jax.experimental.pallas import tpu as pltpu
from jax.experimental.pallas import tpu_sc as plsc

N = 10000
D = 128
E0 = 320000
E = 2 * E0
NC = 2
NS = 16
NW = NC * NS
CHUNK = 112
STEPS = 179
GS = 24
NB = 3
EP = NW * STEPS * CHUNK
PAD = EP - E
NDUMP = 112
NP = N + NDUMP
RPT = NP // NS
RB = 400

_mesh = plsc.VectorSubcoreMesh(core_axis_name="c", subcore_axis_name="s")


def _wid():
    return lax.axis_index("c") * NS + lax.axis_index("s")


@functools.partial(
    pl.kernel,
    out_type=jax.ShapeDtypeStruct((NC, NP, D), jnp.float32),
    mesh=_mesh,
    scratch_types=[
        pltpu.VMEM((STEPS, CHUNK), jnp.int32),
        pltpu.VMEM((CHUNK, D), jnp.float32),
        pltpu.VMEM_SHARED((NP, D), jnp.float32),
        pltpu.SemaphoreType.DMA,
        pltpu.SemaphoreType.DMA,
    ],
    name="gcn_deg_sc",
)
def _deg_sc(dst_hbm, ones_hbm, zeros_hbm, out_hbm, didx, ones_v, acc, c0, c1):
    c = lax.axis_index("c")
    t = lax.axis_index("s")
    w = _wid()
    pltpu.sync_copy(dst_hbm.at[w], didx)
    pltpu.sync_copy(ones_hbm, ones_v)
    pltpu.sync_copy(zeros_hbm, acc.at[pl.ds(t * RPT, RPT)])
    plsc.subcore_barrier()

    def wait_c(sem):
        pltpu.make_async_copy(ones_v, acc.at[didx.at[0]], sem).wait()

    def body(i, carry):
        j0 = 2 * i
        pltpu.async_copy(ones_v, acc.at[didx.at[j0]], c0, add=True)
        pltpu.async_copy(ones_v, acc.at[didx.at[j0 + 1]], c1, add=True)
        wait_c(c0)
        wait_c(c1)
        return carry

    lax.fori_loop(0, STEPS // 2, body, 0, unroll=False)
    for j in range(STEPS - (STEPS % 2), STEPS):
        pltpu.sync_copy(ones_v, acc.at[didx.at[j]], add=True)
    plsc.subcore_barrier()
    pltpu.sync_copy(acc.at[pl.ds(t * RPT, RPT)], out_hbm.at[c, pl.ds(t * RPT, RPT)])


@functools.partial(
    pl.kernel,
    out_type=jax.ShapeDtypeStruct((NC, NP, D), jnp.float32),
    mesh=_mesh,
    scratch_types=[
        pltpu.VMEM((GS, CHUNK), jnp.int32),
        pltpu.VMEM((GS, CHUNK), jnp.int32),
        pltpu.VMEM((CHUNK, D), jnp.float32),
        pltpu.VMEM((CHUNK, D), jnp.float32),
        pltpu.VMEM((CHUNK, D), jnp.float32),
        pltpu.VMEM_SHARED((NP, D), jnp.float32),
        pltpu.SemaphoreType.DMA,
        pltpu.SemaphoreType.DMA,
        pltpu.SemaphoreType.DMA,
        pltpu.SemaphoreType.DMA,
        pltpu.SemaphoreType.DMA,
        pltpu.SemaphoreType.DMA,
    ],
    name="gcn_agg_sc",
)
def _agg_sc(u_hbm, src_hbm, dst_hbm, zeros_hbm, out_hbm,
            sidx, didx, r0, r1, r2, acc, g0, g1, g2, c0, c1, c2):
    c = lax.axis_index("c")
    t = lax.axis_index("s")
    w = _wid()
    rows = (r0, r1, r2)
    gsem = (g0, g1, g2)
    csem = (c0, c1, c2)
    pltpu.sync_copy(zeros_hbm, acc.at[pl.ds(t * RPT, RPT)])
    plsc.subcore_barrier()

    def wait_g(b):
        pltpu.make_async_copy(u_hbm.at[sidx.at[0]], rows[b], gsem[b]).wait()

    def wait_c(b):
        pltpu.make_async_copy(rows[b], acc.at[didx.at[0]], csem[b]).wait()

    def run_group(start, n):
        pltpu.sync_copy(src_hbm.at[w, pl.ds(start, n)], sidx.at[pl.ds(0, n)])
        pltpu.sync_copy(dst_hbm.at[w, pl.ds(start, n)], didx.at[pl.ds(0, n)])
        nt, r = n // 3, n % 3
        for b in range(min(3, n)):
            pltpu.async_copy(u_hbm.at[sidx.at[b]], rows[b], gsem[b])

        def body(k, carry):
            j = 3 * k
            for b in range(3):
                wait_g(b)
                pltpu.async_copy(rows[b], acc.at[didx.at[j + b]], csem[b],
                                 add=True)
            for b in range(3):
                @pl.when(j + 3 + b < n)
                def _(b=b, j=j):
                    wait_c(b)
                    pltpu.async_copy(u_hbm.at[sidx.at[j + 3 + b]], rows[b],
                                     gsem[b])
            return carry

        lax.fori_loop(0, nt, body, 0, unroll=False)
        for i in range(r):
            wait_g(i)
            pltpu.sync_copy(rows[i], acc.at[didx.at[3 * nt + i]], add=True)
        for b in range(3):
            drains = nt - sum(1 for k in range(nt) if 3 * k + 3 + b < n)
            for _ in range(drains):
                wait_c(b)

    off = 0
    while off < STEPS:
        n = min(GS, STEPS - off)
        run_group(off, n)
        off += n
    plsc.subcore_barrier()
    pltpu.sync_copy(acc.at[pl.ds(t * RPT, RPT)], out_hbm.at[c, pl.ds(t * RPT, RPT)])


def _dinv(degp_ref):
    deg = degp_ref[0, :, 0] + degp_ref[1, :, 0] + 1.0
    return lax.rsqrt(deg)


def _mm(a, b):
    return lax.dot_general(a, b, (((1,), (1,)), ((), ())),
                           preferred_element_type=jnp.float32)


def _tc1_body(degp_ref, x_ref, w_ref, o_ref, dv_ref):
    dinv = _dinv(degp_ref)
    o_ref[...] = _mm(x_ref[...], w_ref[...]) * dinv[:, None]
    dv_ref[...] = dinv[:, None] * jnp.ones((1, 16), jnp.float32)


def _tc2_body(dv_ref, p0_ref, p1_ref, u_ref, b_ref, w_ref, o_ref):
    dinv = dv_ref[:, 0]
    agg = p0_ref[0] + p1_ref[0] + u_ref[...]
    h = jnp.maximum(agg * dinv[:, None] + b_ref[...], 0.0)
    o_ref[...] = _mm(h, w_ref[...]) * dinv[:, None]


def _tc3_body(dv_ref, p0_ref, p1_ref, u_ref, b_ref, o_ref):
    dinv = dv_ref[:, 0]
    agg = p0_ref[0] + p1_ref[0] + u_ref[...]
    o_ref[...] = agg * dinv[:, None] + b_ref[...]


_deg_spec = pl.BlockSpec((2, RB, D), lambda i: (0, i, 0))
_dinv_spec = pl.BlockSpec((RB, 16), lambda i: (i, 0))
_p0_spec = pl.BlockSpec((1, RB, D), lambda i: (0, i, 0))
_p1_spec = pl.BlockSpec((1, RB, D), lambda i: (1, i, 0))
_row_spec = pl.BlockSpec((RB, D), lambda i: (i, 0))
_full_spec = pl.BlockSpec((D, D), lambda i: (0, 0))
_bias_spec = pl.BlockSpec((1, D), lambda i: (0, 0))
_grid = (N // RB,)
_out_rows = jax.ShapeDtypeStruct((N, D), jnp.float32)
_parallel = pltpu.CompilerParams(
    dimension_semantics=("arbitrary",))

_tc1 = pl.pallas_call(
    _tc1_body, grid=_grid,
    in_specs=[_deg_spec, _row_spec, _full_spec],
    out_specs=[_row_spec, _dinv_spec],
    out_shape=[_out_rows, jax.ShapeDtypeStruct((N, 16), jnp.float32)],
    compiler_params=_parallel)

_tc2 = pl.pallas_call(
    _tc2_body, grid=_grid,
    in_specs=[_dinv_spec, _p0_spec, _p1_spec, _row_spec, _bias_spec, _full_spec],
    out_specs=_row_spec, out_shape=_out_rows, compiler_params=_parallel)

_tc3 = pl.pallas_call(
    _tc3_body, grid=_grid,
    in_specs=[_dinv_spec, _p0_spec, _p1_spec, _row_spec, _bias_spec],
    out_specs=_row_spec, out_shape=_out_rows, compiler_params=_parallel)


def kernel(x, edge_index, num_nodes, W1, b1, W2, b2):
    ei = edge_index.astype(jnp.int32)
    r, c = ei[0], ei[1]
    ar = jnp.arange(PAD, dtype=jnp.int32)
    pad_dst = N + (ar % NDUMP)
    pad_src = ar % N
    dst = jnp.concatenate([r, c, pad_dst]).reshape(NW, STEPS, CHUNK)
    src = jnp.concatenate([c, r, pad_src]).reshape(NW, STEPS, CHUNK)

    onesD = jnp.ones((CHUNK, D), jnp.float32)
    zerosD = jnp.zeros((RPT, D), jnp.float32)
    b1r = b1.reshape(1, D)
    b2r = b2.reshape(1, D)

    degp = _deg_sc(dst, onesD, zerosD)
    u1, dv = _tc1(degp, x, W1)
    p1 = _agg_sc(u1, src, dst, zerosD)
    u2 = _tc2(dv, p1, p1, u1, b1r, W2)
    p2 = _agg_sc(u2, src, dst, zerosD)
    out = _tc3(dv, p2, p2, u2, b2r)
    return out


# speedup vs baseline: 1.0147x; 1.0147x over previous
import functools

import jax
import jax.numpy as jnp
from jax import lax
from jax.experimental import pallas as pl
from jax.experimental.pallas import tpu as pltpu
from jax.experimental.pallas import tpu_sc as plsc

N = 10000
D = 128
E0 = 320000
E = 2 * E0
NC = 2
NS = 16
NW = NC * NS
CHUNK = 112
STEPS = 179
GS = 24
NB = 3
EP = NW * STEPS * CHUNK
PAD = EP - E
NDUMP = 112
NP = N + NDUMP
RPT = NP // NS
RB = 400

_mesh = plsc.VectorSubcoreMesh(core_axis_name="c", subcore_axis_name="s")


def _wid():
    return lax.axis_index("c") * NS + lax.axis_index("s")


@functools.partial(
    pl.kernel,
    out_type=jax.ShapeDtypeStruct((NC, NP, D), jnp.float32),
    mesh=_mesh,
    scratch_types=[
        pltpu.VMEM((STEPS, CHUNK), jnp.int32),
        pltpu.VMEM((CHUNK, D), jnp.float32),
        pltpu.VMEM_SHARED((NP, D), jnp.float32),
        pltpu.SemaphoreType.DMA,
        pltpu.SemaphoreType.DMA,
    ],
    name="gcn_deg_sc",
)
def _deg_sc(dst_hbm, ones_hbm, zeros_hbm, out_hbm, didx, ones_v, acc, c0, c1):
    c = lax.axis_index("c")
    t = lax.axis_index("s")
    w = _wid()
    pltpu.sync_copy(dst_hbm.at[w], didx)
    pltpu.sync_copy(ones_hbm, ones_v)
    pltpu.sync_copy(zeros_hbm, acc.at[pl.ds(t * RPT, RPT)])
    plsc.subcore_barrier()

    def wait_c(sem):
        pltpu.make_async_copy(ones_v, acc.at[didx.at[0]], sem).wait()

    def body(i, carry):
        j0 = 2 * i
        pltpu.async_copy(ones_v, acc.at[didx.at[j0]], c0, add=True)
        pltpu.async_copy(ones_v, acc.at[didx.at[j0 + 1]], c1, add=True)
        wait_c(c0)
        wait_c(c1)
        return carry

    lax.fori_loop(0, STEPS // 2, body, 0, unroll=False)
    for j in range(STEPS - (STEPS % 2), STEPS):
        pltpu.sync_copy(ones_v, acc.at[didx.at[j]], add=True)
    plsc.subcore_barrier()
    pltpu.sync_copy(acc.at[pl.ds(t * RPT, RPT)], out_hbm.at[c, pl.ds(t * RPT, RPT)])


@functools.partial(
    pl.kernel,
    out_type=jax.ShapeDtypeStruct((NC, NP, D), jnp.float32),
    mesh=_mesh,
    scratch_types=[
        pltpu.VMEM((GS, CHUNK), jnp.int32),
        pltpu.VMEM((GS, CHUNK), jnp.int32),
        pltpu.VMEM((CHUNK, D), jnp.float32),
        pltpu.VMEM((CHUNK, D), jnp.float32),
        pltpu.VMEM((CHUNK, D), jnp.float32),
        pltpu.VMEM_SHARED((NP, D), jnp.float32),
        pltpu.SemaphoreType.DMA,
        pltpu.SemaphoreType.DMA,
        pltpu.SemaphoreType.DMA,
        pltpu.SemaphoreType.DMA,
        pltpu.SemaphoreType.DMA,
        pltpu.SemaphoreType.DMA,
        pltpu.SemaphoreType.DMA,
        pltpu.SemaphoreType.DMA,
        pltpu.SemaphoreType.DMA,
    ],
    name="gcn_agg_sc",
)
def _agg_sc(u_hbm, src_hbm, dst_hbm, zeros_hbm, out_hbm,
            sidx, didx, r0, r1, r2, acc, g0, g1, g2, h0, h1, h2, c0, c1, c2):
    c = lax.axis_index("c")
    t = lax.axis_index("s")
    w = _wid()
    rows = (r0, r1, r2)
    gsem = (g0, g1, g2)
    hsem = (h0, h1, h2)
    csem = (c0, c1, c2)
    HA = CHUNK // 2
    pltpu.sync_copy(zeros_hbm, acc.at[pl.ds(t * RPT, RPT)])
    plsc.subcore_barrier()

    def issue_g(b, j):
        pltpu.async_copy(u_hbm.at[sidx.at[j, pl.ds(0, HA)]],
                         rows[b].at[pl.ds(0, HA)], gsem[b])
        pltpu.async_copy(u_hbm.at[sidx.at[j, pl.ds(HA, HA)]],
                         rows[b].at[pl.ds(HA, HA)], hsem[b])

    def wait_g(b):
        pltpu.make_async_copy(u_hbm.at[sidx.at[0, pl.ds(0, HA)]],
                              rows[b].at[pl.ds(0, HA)], gsem[b]).wait()
        pltpu.make_async_copy(u_hbm.at[sidx.at[0, pl.ds(0, HA)]],
                              rows[b].at[pl.ds(HA, HA)], hsem[b]).wait()

    def wait_c(b):
        pltpu.make_async_copy(rows[b], acc.at[didx.at[0]], csem[b]).wait()

    def run_group(start, n):
        pltpu.sync_copy(src_hbm.at[w, pl.ds(start, n)], sidx.at[pl.ds(0, n)])
        pltpu.sync_copy(dst_hbm.at[w, pl.ds(start, n)], didx.at[pl.ds(0, n)])
        nt, r = n // 3, n % 3
        for b in range(min(3, n)):
            issue_g(b, b)

        def body(k, carry):
            j = 3 * k
            for b in range(3):
                wait_g(b)
                pltpu.async_copy(rows[b], acc.at[didx.at[j + b]], csem[b],
                                 add=True)
            for b in range(3):
                @pl.when(j + 3 + b < n)
                def _(b=b, j=j):
                    wait_c(b)
                    issue_g(b, j + 3 + b)
            return carry

        lax.fori_loop(0, nt, body, 0, unroll=False)
        for i in range(r):
            wait_g(i)
            pltpu.sync_copy(rows[i], acc.at[didx.at[3 * nt + i]], add=True)
        for b in range(3):
            drains = nt - sum(1 for k in range(nt) if 3 * k + 3 + b < n)
            for _ in range(drains):
                wait_c(b)

    off = 0
    while off < STEPS:
        n = min(GS, STEPS - off)
        run_group(off, n)
        off += n
    plsc.subcore_barrier()
    pltpu.sync_copy(acc.at[pl.ds(t * RPT, RPT)], out_hbm.at[c, pl.ds(t * RPT, RPT)])


def _dinv(degp_ref):
    deg = degp_ref[0, :, 0] + degp_ref[1, :, 0] + 1.0
    return lax.rsqrt(deg)


def _mm(a, b):
    return lax.dot_general(a, b, (((1,), (1,)), ((), ())),
                           preferred_element_type=jnp.float32)


def _tc1_body(degp_ref, x_ref, w_ref, o_ref, dv_ref):
    dinv = _dinv(degp_ref)
    o_ref[...] = _mm(x_ref[...], w_ref[...]) * dinv[:, None]
    dv_ref[...] = dinv[:, None] * jnp.ones((1, 16), jnp.float32)


def _tc2_body(dv_ref, p0_ref, p1_ref, u_ref, b_ref, w_ref, o_ref):
    dinv = dv_ref[:, 0]
    agg = p0_ref[0] + p1_ref[0] + u_ref[...]
    h = jnp.maximum(agg * dinv[:, None] + b_ref[...], 0.0)
    o_ref[...] = _mm(h, w_ref[...]) * dinv[:, None]


def _tc3_body(dv_ref, p0_ref, p1_ref, u_ref, b_ref, o_ref):
    dinv = dv_ref[:, 0]
    agg = p0_ref[0] + p1_ref[0] + u_ref[...]
    o_ref[...] = agg * dinv[:, None] + b_ref[...]


_deg_spec = pl.BlockSpec((2, RB, D), lambda i: (0, i, 0))
_dinv_spec = pl.BlockSpec((RB, 16), lambda i: (i, 0))
_p0_spec = pl.BlockSpec((1, RB, D), lambda i: (0, i, 0))
_p1_spec = pl.BlockSpec((1, RB, D), lambda i: (1, i, 0))
_row_spec = pl.BlockSpec((RB, D), lambda i: (i, 0))
_full_spec = pl.BlockSpec((D, D), lambda i: (0, 0))
_bias_spec = pl.BlockSpec((1, D), lambda i: (0, 0))
_grid = (N // RB,)
_out_rows = jax.ShapeDtypeStruct((N, D), jnp.float32)
_parallel = pltpu.CompilerParams(
    dimension_semantics=("arbitrary",))

_tc1 = pl.pallas_call(
    _tc1_body, grid=_grid,
    in_specs=[_deg_spec, _row_spec, _full_spec],
    out_specs=[_row_spec, _dinv_spec],
    out_shape=[_out_rows, jax.ShapeDtypeStruct((N, 16), jnp.float32)],
    compiler_params=_parallel)

_tc2 = pl.pallas_call(
    _tc2_body, grid=_grid,
    in_specs=[_dinv_spec, _p0_spec, _p1_spec, _row_spec, _bias_spec, _full_spec],
    out_specs=_row_spec, out_shape=_out_rows, compiler_params=_parallel)

_tc3 = pl.pallas_call(
    _tc3_body, grid=_grid,
    in_specs=[_dinv_spec, _p0_spec, _p1_spec, _row_spec, _bias_spec],
    out_specs=_row_spec, out_shape=_out_rows, compiler_params=_parallel)


def kernel(x, edge_index, num_nodes, W1, b1, W2, b2):
    ei = edge_index.astype(jnp.int32)
    r, c = ei[0], ei[1]
    ar = jnp.arange(PAD, dtype=jnp.int32)
    pad_dst = N + (ar % NDUMP)
    pad_src = ar % N
    dst = jnp.concatenate([r, c, pad_dst]).reshape(NW, STEPS, CHUNK)
    src = jnp.concatenate([c, r, pad_src]).reshape(NW, STEPS, CHUNK)

    onesD = jnp.ones((CHUNK, D), jnp.float32)
    zerosD = jnp.zeros((RPT, D), jnp.float32)
    b1r = b1.reshape(1, D)
    b2r = b2.reshape(1, D)

    degp = _deg_sc(dst, onesD, zerosD)
    u1, dv = _tc1(degp, x, W1)
    p1 = _agg_sc(u1, src, dst, zerosD)
    u2 = _tc2(dv, p1, p1, u1, b1r, W2)
    p2 = _agg_sc(u2, src, dst, zerosD)
    out = _tc3(dv, p2, p2, u2, b2r)
    return out


# speedup vs baseline: 1.0712x; 1.0556x over previous
import functools

import jax
import jax.numpy as jnp
from jax import lax
from jax.experimental import pallas as pl
from jax.experimental.pallas import tpu as pltpu
from jax.experimental.pallas import tpu_sc as plsc

N = 10000
D = 128
E0 = 320000
E = 2 * E0
NC = 2
NS = 16
NW = NC * NS
CHUNK = 112
STEPS = 179
GS = 24
NB = 3
EP = NW * STEPS * CHUNK
PAD = EP - E
NDUMP = 112
NP = N + NDUMP
RPT = NP // NS
RB = 400

_mesh = plsc.VectorSubcoreMesh(core_axis_name="c", subcore_axis_name="s")


def _wid():
    return lax.axis_index("c") * NS + lax.axis_index("s")


@functools.partial(
    pl.kernel,
    out_type=jax.ShapeDtypeStruct((NC, NP, D), jnp.float32),
    mesh=_mesh,
    scratch_types=[
        pltpu.VMEM((STEPS, CHUNK), jnp.int32),
        pltpu.VMEM((CHUNK, D), jnp.float32),
        pltpu.VMEM_SHARED((NP, D), jnp.float32),
        pltpu.SemaphoreType.DMA,
        pltpu.SemaphoreType.DMA,
    ],
    name="gcn_deg_sc",
)
def _deg_sc(dst_hbm, ones_hbm, zeros_hbm, out_hbm, didx, ones_v, acc, c0, c1):
    c = lax.axis_index("c")
    t = lax.axis_index("s")
    w = _wid()
    pltpu.sync_copy(dst_hbm.at[w], didx)
    pltpu.sync_copy(ones_hbm, ones_v)
    pltpu.sync_copy(zeros_hbm, acc.at[pl.ds(t * RPT, RPT)])
    plsc.subcore_barrier()

    def wait_c(sem):
        pltpu.make_async_copy(ones_v, acc.at[didx.at[0]], sem).wait()

    def body(i, carry):
        j0 = 2 * i
        pltpu.async_copy(ones_v, acc.at[didx.at[j0]], c0, add=True)
        pltpu.async_copy(ones_v, acc.at[didx.at[j0 + 1]], c1, add=True)
        wait_c(c0)
        wait_c(c1)
        return carry

    lax.fori_loop(0, STEPS // 2, body, 0, unroll=False)
    for j in range(STEPS - (STEPS % 2), STEPS):
        pltpu.sync_copy(ones_v, acc.at[didx.at[j]], add=True)
    plsc.subcore_barrier()
    pltpu.sync_copy(acc.at[pl.ds(t * RPT, RPT)], out_hbm.at[c, pl.ds(t * RPT, RPT)])


@functools.partial(
    pl.kernel,
    out_type=jax.ShapeDtypeStruct((NC, NP, D), jnp.float32),
    mesh=_mesh,
    scratch_types=[
        pltpu.VMEM((GS, CHUNK), jnp.int32),
        pltpu.VMEM((GS, CHUNK), jnp.int32),
        pltpu.VMEM((CHUNK, D), jnp.float32),
        pltpu.VMEM((CHUNK, D), jnp.float32),
        pltpu.VMEM((CHUNK, D), jnp.float32),
        pltpu.VMEM_SHARED((NP, D), jnp.float32),
    ] + [pltpu.SemaphoreType.DMA] * 15,
    name="gcn_agg_sc",
)
def _agg_sc(u_hbm, src_hbm, dst_hbm, zeros_hbm, out_hbm,
            sidx, didx, r0, r1, r2, acc, *sems):
    c = lax.axis_index("c")
    t = lax.axis_index("s")
    w = _wid()
    rows = (r0, r1, r2)
    gsem = (sems[0:4], sems[4:8], sems[8:12])
    csem = sems[12:15]
    SPLITS = ((0, 32), (32, 32), (64, 24), (88, 24))
    pltpu.sync_copy(zeros_hbm, acc.at[pl.ds(t * RPT, RPT)])
    plsc.subcore_barrier()

    def issue_g(b, j):
        for k, (o, sz) in enumerate(SPLITS):
            pltpu.async_copy(u_hbm.at[sidx.at[j, pl.ds(o, sz)]],
                             rows[b].at[pl.ds(o, sz)], gsem[b][k])

    def wait_g(b):
        for k, (o, sz) in enumerate(SPLITS):
            pltpu.make_async_copy(u_hbm.at[sidx.at[0, pl.ds(o, sz)]],
                                  rows[b].at[pl.ds(o, sz)], gsem[b][k]).wait()

    def wait_c(b):
        pltpu.make_async_copy(rows[b], acc.at[didx.at[0]], csem[b]).wait()

    def run_group(start, n):
        pltpu.sync_copy(src_hbm.at[w, pl.ds(start, n)], sidx.at[pl.ds(0, n)])
        pltpu.sync_copy(dst_hbm.at[w, pl.ds(start, n)], didx.at[pl.ds(0, n)])
        nt, r = n // 3, n % 3
        for b in range(min(3, n)):
            issue_g(b, b)

        def body(k, carry):
            j = 3 * k
            for b in range(3):
                wait_g(b)
                pltpu.async_copy(rows[b], acc.at[didx.at[j + b]], csem[b],
                                 add=True)
            for b in range(3):
                @pl.when(j + 3 + b < n)
                def _(b=b, j=j):
                    wait_c(b)
                    issue_g(b, j + 3 + b)
            return carry

        lax.fori_loop(0, nt, body, 0, unroll=False)
        for i in range(r):
            wait_g(i)
            pltpu.sync_copy(rows[i], acc.at[didx.at[3 * nt + i]], add=True)
        for b in range(3):
            drains = nt - sum(1 for k in range(nt) if 3 * k + 3 + b < n)
            for _ in range(drains):
                wait_c(b)

    off = 0
    while off < STEPS:
        n = min(GS, STEPS - off)
        run_group(off, n)
        off += n
    plsc.subcore_barrier()
    pltpu.sync_copy(acc.at[pl.ds(t * RPT, RPT)], out_hbm.at[c, pl.ds(t * RPT, RPT)])


def _dinv(degp_ref):
    deg = degp_ref[0, :, 0] + degp_ref[1, :, 0] + 1.0
    return lax.rsqrt(deg)


def _mm(a, b):
    return lax.dot_general(a, b, (((1,), (1,)), ((), ())),
                           preferred_element_type=jnp.float32)


def _tc1_body(degp_ref, x_ref, w_ref, o_ref, dv_ref):
    dinv = _dinv(degp_ref)
    o_ref[...] = _mm(x_ref[...], w_ref[...]) * dinv[:, None]
    dv_ref[...] = dinv[:, None] * jnp.ones((1, 16), jnp.float32)


def _tc2_body(dv_ref, p0_ref, p1_ref, u_ref, b_ref, w_ref, o_ref):
    dinv = dv_ref[:, 0]
    agg = p0_ref[0] + p1_ref[0] + u_ref[...]
    h = jnp.maximum(agg * dinv[:, None] + b_ref[...], 0.0)
    o_ref[...] = _mm(h, w_ref[...]) * dinv[:, None]


def _tc3_body(dv_ref, p0_ref, p1_ref, u_ref, b_ref, o_ref):
    dinv = dv_ref[:, 0]
    agg = p0_ref[0] + p1_ref[0] + u_ref[...]
    o_ref[...] = agg * dinv[:, None] + b_ref[...]


_deg_spec = pl.BlockSpec((2, RB, D), lambda i: (0, i, 0))
_dinv_spec = pl.BlockSpec((RB, 16), lambda i: (i, 0))
_p0_spec = pl.BlockSpec((1, RB, D), lambda i: (0, i, 0))
_p1_spec = pl.BlockSpec((1, RB, D), lambda i: (1, i, 0))
_row_spec = pl.BlockSpec((RB, D), lambda i: (i, 0))
_full_spec = pl.BlockSpec((D, D), lambda i: (0, 0))
_bias_spec = pl.BlockSpec((1, D), lambda i: (0, 0))
_grid = (N // RB,)
_out_rows = jax.ShapeDtypeStruct((N, D), jnp.float32)
_parallel = pltpu.CompilerParams(
    dimension_semantics=("arbitrary",))

_tc1 = pl.pallas_call(
    _tc1_body, grid=_grid,
    in_specs=[_deg_spec, _row_spec, _full_spec],
    out_specs=[_row_spec, _dinv_spec],
    out_shape=[_out_rows, jax.ShapeDtypeStruct((N, 16), jnp.float32)],
    compiler_params=_parallel)

_tc2 = pl.pallas_call(
    _tc2_body, grid=_grid,
    in_specs=[_dinv_spec, _p0_spec, _p1_spec, _row_spec, _bias_spec, _full_spec],
    out_specs=_row_spec, out_shape=_out_rows, compiler_params=_parallel)

_tc3 = pl.pallas_call(
    _tc3_body, grid=_grid,
    in_specs=[_dinv_spec, _p0_spec, _p1_spec, _row_spec, _bias_spec],
    out_specs=_row_spec, out_shape=_out_rows, compiler_params=_parallel)


def kernel(x, edge_index, num_nodes, W1, b1, W2, b2):
    ei = edge_index.astype(jnp.int32)
    r, c = ei[0], ei[1]
    ar = jnp.arange(PAD, dtype=jnp.int32)
    pad_dst = N + (ar % NDUMP)
    pad_src = ar % N
    dst = jnp.concatenate([r, c, pad_dst]).reshape(NW, STEPS, CHUNK)
    src = jnp.concatenate([c, r, pad_src]).reshape(NW, STEPS, CHUNK)

    onesD = jnp.ones((CHUNK, D), jnp.float32)
    zerosD = jnp.zeros((RPT, D), jnp.float32)
    b1r = b1.reshape(1, D)
    b2r = b2.reshape(1, D)

    degp = _deg_sc(dst, onesD, zerosD)
    u1, dv = _tc1(degp, x, W1)
    p1 = _agg_sc(u1, src, dst, zerosD)
    u2 = _tc2(dv, p1, p1, u1, b1r, W2)
    p2 = _agg_sc(u2, src, dst, zerosD)
    out = _tc3(dv, p2, p2, u2, b2r)
    return out
